# baseline (speedup 1.0000x reference)
import jax
import jax.numpy as jnp
from jax import lax
from jax.experimental import pallas as pl
from jax.experimental.pallas import tpu as pltpu
from jax.experimental.pallas import tpu_sc as plsc

N = 10000
E = 320000
D = 128

NC = 2
NS = 16
NW = NC * NS

CHUNK = 128

K3 = 80
KPP = 160
SLOW_CID = 1
KA = 40
KB = KPP - KA
BSTG = 40
EPAD = NS * KPP * CHUNK

NPAD = 10240
WPT = NPAD // NS
NBUF = 2

BN = 512


def _mesh():
    return plsc.VectorSubcoreMesh(
        core_axis_name="c", subcore_axis_name="s", num_cores=NC, num_subcores=NS
    )


def _deg_body(col_hbm, out_hbm, col2d, degv):
    cid = lax.axis_index("c")
    sid = lax.axis_index("s")
    wid = cid * NS + sid

    zeros16 = jnp.zeros((16,), jnp.float32)
    ones16 = jnp.ones((16,), jnp.float32)

    def _fz(r, _):
        degv[pl.ds(r * 16, 16)] = zeros16
        return 0

    lax.fori_loop(0, NPAD // 16, _fz, 0)

    pltpu.sync_copy(col_hbm.at[pl.ds(wid * K3, K3)], col2d)

    def _step(j, _):
        for l in range(CHUNK // 16):
            c16 = col2d[j, pl.ds(l * 16, 16)]
            plsc.addupdate_scatter(degv, [c16], ones16)
        return 0

    lax.fori_loop(0, K3, _step, 0)

    pltpu.sync_copy(degv, out_hbm.at[pl.ds(wid * NPAD, NPAD)])


def _deg_call(col):
    k = pl.kernel(
        _deg_body,
        out_type=jax.ShapeDtypeStruct((NW * NPAD,), jnp.float32),
        mesh=_mesh(),
        compiler_params=pltpu.CompilerParams(needs_layout_passes=False),
        scratch_types=[
            pltpu.VMEM((K3, CHUNK), jnp.int32),
            pltpu.VMEM((NPAD,), jnp.float32),
        ],
    )
    return k(col)


def _agg_body(g_hbm, row_hbm, col_hbm, out_hbm, row2d, col2d, rows_v,
              acc_sh, gsem, ssem):
    cid = lax.axis_index("c")
    sid = lax.axis_index("s")

    zeros16 = jnp.zeros((16,), jnp.float32)

    def _fz(r, _):
        for c in range(D // 16):
            rows_v[r, pl.ds(c * 16, 16)] = zeros16
        return 0

    lax.fori_loop(0, CHUNK, _fz, 0)

    for t in range(WPT // CHUNK):
        pltpu.sync_copy(rows_v.at[pl.ds(0, CHUNK)],
                        acc_sh.at[pl.ds(sid * WPT + t * CHUNK, CHUNK)])
    plsc.subcore_barrier()

    def _gather_start(j):
        pltpu.async_copy(g_hbm.at[row2d.at[j]],
                         rows_v.at[pl.ds((j % NBUF) * CHUNK, CHUNK)], gsem)

    def _gather_wait(j):
        pltpu.make_async_copy(
            g_hbm.at[row2d.at[j]],
            rows_v.at[pl.ds((j % NBUF) * CHUNK, CHUNK)], gsem).wait()

    def _scatter_start(j):
        pltpu.async_copy(rows_v.at[pl.ds((j % NBUF) * CHUNK, CHUNK)],
                         acc_sh.at[col2d.at[j]], ssem, add=True)

    def _scatter_wait(j):
        pltpu.make_async_copy(
            rows_v.at[pl.ds((j % NBUF) * CHUNK, CHUNK)],
            acc_sh.at[col2d.at[j]], ssem).wait()

    is_slow = cid == SLOW_CID
    base = sid * KPP + jnp.where(is_slow, 0, KA)
    nblk = jnp.where(is_slow, KA // BSTG, KB // BSTG)

    def _phase(p, _):
        blk = base + p * BSTG
        pltpu.sync_copy(row_hbm.at[pl.ds(blk, BSTG)], row2d)
        pltpu.sync_copy(col_hbm.at[pl.ds(blk, BSTG)], col2d)

        for m in range(NBUF - 1):
            _gather_start(m)

        def _step(j, _):
            _gather_wait(j)

            @pl.when(j >= 1)
            def _():
                _scatter_wait(j - 1)

            @pl.when(j + NBUF - 1 < BSTG)
            def _():
                _gather_start(j + NBUF - 1)

            _scatter_start(j)
            return 0

        lax.fori_loop(0, BSTG, _step, 0)
        _scatter_wait(BSTG - 1)
        return 0

    lax.fori_loop(0, nblk, _phase, 0)

    plsc.subcore_barrier()

    for t in range(WPT // CHUNK):
        r0 = sid * WPT + t * CHUNK
        pltpu.sync_copy(acc_sh.at[pl.ds(r0, CHUNK)],
                        out_hbm.at[pl.ds(cid * NPAD + r0, CHUNK)])


def _agg_call(g, row, col):
    k = pl.kernel(
        _agg_body,
        out_type=jax.ShapeDtypeStruct((NC * NPAD, D), jnp.float32),
        mesh=_mesh(),
        scratch_types=[
            pltpu.VMEM((BSTG, CHUNK), jnp.int32),
            pltpu.VMEM((BSTG, CHUNK), jnp.int32),
            pltpu.VMEM((NBUF * CHUNK, D), jnp.float32),
            pltpu.VMEM_SHARED((NPAD, D), jnp.float32),
            pltpu.SemaphoreType.DMA,
            pltpu.SemaphoreType.DMA,
        ],
    )
    return k(g, row, col)


def _pre_body(x_ref, w_ref, dt_ref, g_ref, dinv_ref):
    deg = jnp.sum(dt_ref[...], axis=1, keepdims=True) + 1.0
    dinv = lax.rsqrt(deg)
    dinv_ref[...] = jnp.broadcast_to(dinv, dinv_ref.shape)
    h = jnp.dot(x_ref[...], w_ref[...], preferred_element_type=jnp.float32)
    g_ref[...] = h * dinv


def _pre_call(x, w, degT):
    return pl.pallas_call(
        _pre_body,
        grid=(NPAD // BN,),
        in_specs=[
            pl.BlockSpec((BN, D), lambda i: (i, 0)),
            pl.BlockSpec((D, D), lambda i: (0, 0)),
            pl.BlockSpec((BN, NW), lambda i: (i, 0)),
        ],
        out_specs=[
            pl.BlockSpec((BN, D), lambda i: (i, 0)),
            pl.BlockSpec((BN, NW), lambda i: (i, 0)),
        ],
        out_shape=[
            jax.ShapeDtypeStruct((NPAD, D), jnp.float32),
            jax.ShapeDtypeStruct((NPAD, NW), jnp.float32),
        ],
    )(x, w, degT)


def _mid_body(p0_ref, p1_ref, g1_ref, dinv_ref, b_ref, w_ref, g2_ref):
    dinv = dinv_ref[...][:, :1]
    x2 = dinv * (p0_ref[...] + p1_ref[...] + g1_ref[...]) + b_ref[...]
    x2 = jnp.maximum(x2, 0.0)
    g2_ref[...] = (
        jnp.dot(x2, w_ref[...], preferred_element_type=jnp.float32) * dinv
    )


def _mid_call(p0, p1, g1, dinv32, b, w):
    return pl.pallas_call(
        _mid_body,
        grid=(NPAD // BN,),
        in_specs=[
            pl.BlockSpec((BN, D), lambda i: (i, 0)),
            pl.BlockSpec((BN, D), lambda i: (i, 0)),
            pl.BlockSpec((BN, D), lambda i: (i, 0)),
            pl.BlockSpec((BN, NW), lambda i: (i, 0)),
            pl.BlockSpec((1, D), lambda i: (0, 0)),
            pl.BlockSpec((D, D), lambda i: (0, 0)),
        ],
        out_specs=pl.BlockSpec((BN, D), lambda i: (i, 0)),
        out_shape=jax.ShapeDtypeStruct((NPAD, D), jnp.float32),
    )(p0, p1, g1, dinv32, b, w)


def _post_body(p0_ref, p1_ref, g2_ref, dinv_ref, b_ref, o_ref):
    dinv = dinv_ref[...][:, :1]
    o_ref[...] = dinv * (p0_ref[...] + p1_ref[...] + g2_ref[...]) + b_ref[...]


def _post_call(p0, p1, g2, dinv32, b):
    return pl.pallas_call(
        _post_body,
        grid=(NPAD // BN,),
        in_specs=[
            pl.BlockSpec((BN, D), lambda i: (i, 0)),
            pl.BlockSpec((BN, D), lambda i: (i, 0)),
            pl.BlockSpec((BN, D), lambda i: (i, 0)),
            pl.BlockSpec((BN, NW), lambda i: (i, 0)),
            pl.BlockSpec((1, D), lambda i: (0, 0)),
        ],
        out_specs=pl.BlockSpec((BN, D), lambda i: (i, 0)),
        out_shape=jax.ShapeDtypeStruct((NPAD, D), jnp.float32),
    )(p0, p1, g2, dinv32, b)


def kernel(node_features, edge_index, W1, b1, W2, b2):
    ei = edge_index.astype(jnp.int32)
    row = jnp.pad(ei[0], (0, EPAD - E), constant_values=N).reshape(NW * K3, CHUNK)
    col = jnp.pad(ei[1], (0, EPAD - E), constant_values=N).reshape(NW * K3, CHUNK)
    x_pad = jnp.pad(node_features, ((0, NPAD - N), (0, 0)))

    deg = _deg_call(col)
    degT = deg.reshape(NW, NPAD).T

    g1, dinv32 = _pre_call(x_pad, W1, degT)
    agg1 = _agg_call(g1, row, col)
    g2 = _mid_call(agg1[:NPAD], agg1[NPAD:], g1, dinv32, b1.reshape(1, D), W2)
    agg2 = _agg_call(g2, row, col)
    out = _post_call(agg2[:NPAD], agg2[NPAD:], g2, dinv32, b2.reshape(1, D))
    return out[:N]


# speedup vs baseline: 2.9993x; 2.9993x over previous
import jax
import jax.numpy as jnp
from jax import lax
from jax.experimental import pallas as pl
from jax.experimental.pallas import tpu as pltpu
from jax.experimental.pallas import tpu_sc as plsc

N = 10000
E = 320000
D = 128

NC = 2
NS = 16
NW = NC * NS

CHUNK = 128

K3 = 80
KPP = 160
SLOW_CID = 0
KA = 80
KB = KPP - KA
BSTG = 40
EPAD = NS * KPP * CHUNK

NPAD = 10240
WPT = NPAD // NS
NBUF = 2

BN = 512


def _mesh():
    return plsc.VectorSubcoreMesh(
        core_axis_name="c", subcore_axis_name="s", num_cores=NC, num_subcores=NS
    )


def _deg_body(col_hbm, out_hbm, col2d, degv):
    cid = lax.axis_index("c")
    sid = lax.axis_index("s")
    wid = cid * NS + sid

    zeros16 = jnp.zeros((16,), jnp.float32)
    ones16 = jnp.ones((16,), jnp.float32)

    def _fz(r, _):
        degv[pl.ds(r * 16, 16)] = zeros16
        return 0

    lax.fori_loop(0, NPAD // 16, _fz, 0)

    pltpu.sync_copy(col_hbm.at[pl.ds(wid * K3, K3)], col2d)

    def _step(j, _):
        for l in range(CHUNK // 16):
            c16 = col2d[j, pl.ds(l * 16, 16)]
            plsc.addupdate_scatter(degv, [c16], ones16)
        return 0

    lax.fori_loop(0, K3, _step, 0)

    pltpu.sync_copy(degv, out_hbm.at[pl.ds(wid * NPAD, NPAD)])


def _deg_call(col):
    k = pl.kernel(
        _deg_body,
        out_type=jax.ShapeDtypeStruct((NW * NPAD,), jnp.float32),
        mesh=_mesh(),
        compiler_params=pltpu.CompilerParams(needs_layout_passes=False),
        scratch_types=[
            pltpu.VMEM((K3, CHUNK), jnp.int32),
            pltpu.VMEM((NPAD,), jnp.float32),
        ],
    )
    return k(col)


def _agg_body(g_hbm, row_hbm, col_hbm, out_hbm, row2d, col2d, rows_v,
              acc_sh, gsem, ssem):
    cid = lax.axis_index("c")
    sid = lax.axis_index("s")

    zeros16 = jnp.zeros((16,), jnp.float32)

    def _fz(r, _):
        for c in range(D // 16):
            rows_v[r, pl.ds(c * 16, 16)] = zeros16
        return 0

    lax.fori_loop(0, CHUNK, _fz, 0)

    for t in range(WPT // CHUNK):
        pltpu.sync_copy(rows_v.at[pl.ds(0, CHUNK)],
                        acc_sh.at[pl.ds(sid * WPT + t * CHUNK, CHUNK)])
    plsc.subcore_barrier()

    def _gather_start(j):
        pltpu.async_copy(g_hbm.at[row2d.at[j]],
                         rows_v.at[pl.ds((j % NBUF) * CHUNK, CHUNK)], gsem)

    def _gather_wait(j):
        pltpu.make_async_copy(
            g_hbm.at[row2d.at[j]],
            rows_v.at[pl.ds((j % NBUF) * CHUNK, CHUNK)], gsem).wait()

    def _scatter_start(j):
        pltpu.async_copy(rows_v.at[pl.ds((j % NBUF) * CHUNK, CHUNK)],
                         acc_sh.at[col2d.at[j]], ssem, add=True)

    def _scatter_wait(j):
        pltpu.make_async_copy(
            rows_v.at[pl.ds((j % NBUF) * CHUNK, CHUNK)],
            acc_sh.at[col2d.at[j]], ssem).wait()

    is_slow = cid == SLOW_CID
    base = sid * KPP + jnp.where(is_slow, 0, KA)
    nblk = jnp.where(is_slow, KA // BSTG, KB // BSTG)

    def _phase(p, _):
        blk = base + p * BSTG
        pltpu.sync_copy(row_hbm.at[pl.ds(blk, BSTG)], row2d)
        pltpu.sync_copy(col_hbm.at[pl.ds(blk, BSTG)], col2d)

        for m in range(NBUF - 1):
            _gather_start(m)

        def _step(j, _):
            _gather_wait(j)

            @pl.when(j >= 1)
            def _():
                _scatter_wait(j - 1)

            @pl.when(j + NBUF - 1 < BSTG)
            def _():
                _gather_start(j + NBUF - 1)

            _scatter_start(j)
            return 0

        lax.fori_loop(0, BSTG, _step, 0)
        _scatter_wait(BSTG - 1)
        return 0

    lax.fori_loop(0, nblk, _phase, 0)

    plsc.subcore_barrier()

    for t in range(WPT // CHUNK):
        r0 = sid * WPT + t * CHUNK
        pltpu.sync_copy(acc_sh.at[pl.ds(r0, CHUNK)],
                        out_hbm.at[pl.ds(cid * NPAD + r0, CHUNK)])


def _agg_call(g, row, col):
    k = pl.kernel(
        _agg_body,
        out_type=jax.ShapeDtypeStruct((NC * NPAD, D), jnp.float32),
        mesh=_mesh(),
        scratch_types=[
            pltpu.VMEM((BSTG, CHUNK), jnp.int32),
            pltpu.VMEM((BSTG, CHUNK), jnp.int32),
            pltpu.VMEM((NBUF * CHUNK, D), jnp.float32),
            pltpu.VMEM_SHARED((NPAD, D), jnp.float32),
            pltpu.SemaphoreType.DMA,
            pltpu.SemaphoreType.DMA,
        ],
    )
    return k(g, row, col)


def _pre_body(x_ref, w_ref, dt_ref, g_ref, dinv_ref):
    deg = jnp.sum(dt_ref[...], axis=1, keepdims=True) + 1.0
    dinv = lax.rsqrt(deg)
    dinv_ref[...] = jnp.broadcast_to(dinv, dinv_ref.shape)
    h = jnp.dot(x_ref[...], w_ref[...], preferred_element_type=jnp.float32)
    g_ref[...] = h * dinv


def _pre_call(x, w, degT):
    return pl.pallas_call(
        _pre_body,
        grid=(NPAD // BN,),
        in_specs=[
            pl.BlockSpec((BN, D), lambda i: (i, 0)),
            pl.BlockSpec((D, D), lambda i: (0, 0)),
            pl.BlockSpec((BN, NW), lambda i: (i, 0)),
        ],
        out_specs=[
            pl.BlockSpec((BN, D), lambda i: (i, 0)),
            pl.BlockSpec((BN, NW), lambda i: (i, 0)),
        ],
        out_shape=[
            jax.ShapeDtypeStruct((NPAD, D), jnp.float32),
            jax.ShapeDtypeStruct((NPAD, NW), jnp.float32),
        ],
    )(x, w, degT)


def _mid_body(p0_ref, p1_ref, g1_ref, dinv_ref, b_ref, w_ref, g2_ref):
    dinv = dinv_ref[...][:, :1]
    x2 = dinv * (p0_ref[...] + p1_ref[...] + g1_ref[...]) + b_ref[...]
    x2 = jnp.maximum(x2, 0.0)
    g2_ref[...] = (
        jnp.dot(x2, w_ref[...], preferred_element_type=jnp.float32) * dinv
    )


def _mid_call(p0, p1, g1, dinv32, b, w):
    return pl.pallas_call(
        _mid_body,
        grid=(NPAD // BN,),
        in_specs=[
            pl.BlockSpec((BN, D), lambda i: (i, 0)),
            pl.BlockSpec((BN, D), lambda i: (i, 0)),
            pl.BlockSpec((BN, D), lambda i: (i, 0)),
            pl.BlockSpec((BN, NW), lambda i: (i, 0)),
            pl.BlockSpec((1, D), lambda i: (0, 0)),
            pl.BlockSpec((D, D), lambda i: (0, 0)),
        ],
        out_specs=pl.BlockSpec((BN, D), lambda i: (i, 0)),
        out_shape=jax.ShapeDtypeStruct((NPAD, D), jnp.float32),
    )(p0, p1, g1, dinv32, b, w)


def _post_body(p0_ref, p1_ref, g2_ref, dinv_ref, b_ref, o_ref):
    dinv = dinv_ref[...][:, :1]
    o_ref[...] = dinv * (p0_ref[...] + p1_ref[...] + g2_ref[...]) + b_ref[...]


def _post_call(p0, p1, g2, dinv32, b):
    return pl.pallas_call(
        _post_body,
        grid=(NPAD // BN,),
        in_specs=[
            pl.BlockSpec((BN, D), lambda i: (i, 0)),
            pl.BlockSpec((BN, D), lambda i: (i, 0)),
            pl.BlockSpec((BN, D), lambda i: (i, 0)),
            pl.BlockSpec((BN, NW), lambda i: (i, 0)),
            pl.BlockSpec((1, D), lambda i: (0, 0)),
        ],
        out_specs=pl.BlockSpec((BN, D), lambda i: (i, 0)),
        out_shape=jax.ShapeDtypeStruct((NPAD, D), jnp.float32),
    )(p0, p1, g2, dinv32, b)


def kernel(node_features, edge_index, W1, b1, W2, b2):
    ei = edge_index.astype(jnp.int32)
    padv = N + (jnp.arange(EPAD - E, dtype=jnp.int32) % (NPAD - N))
    row = jnp.concatenate([ei[0], padv]).reshape(NW * K3, CHUNK)
    col = jnp.concatenate([ei[1], padv]).reshape(NW * K3, CHUNK)
    x_pad = jnp.pad(node_features, ((0, NPAD - N), (0, 0)))

    deg = _deg_call(col)
    degT = deg.reshape(NW, NPAD).T

    g1, dinv32 = _pre_call(x_pad, W1, degT)
    agg1 = _agg_call(g1, row, col)
    g2 = _mid_call(agg1[:NPAD], agg1[NPAD:], g1, dinv32, b1.reshape(1, D), W2)
    agg2 = _agg_call(g2, row, col)
    out = _post_call(agg2[:NPAD], agg2[NPAD:], g2, dinv32, b2.reshape(1, D))
    return out[:N]


# speedup vs baseline: 3.1243x; 1.0417x over previous
import jax
import jax.numpy as jnp
from jax import lax
from jax.experimental import pallas as pl
from jax.experimental.pallas import tpu as pltpu
from jax.experimental.pallas import tpu_sc as plsc

N = 10000
E = 320000
D = 128

NC = 2
NS = 16
NW = NC * NS

CHUNK = 128

K3 = 80
KPP = 160
SLOW_CID = 0
KA = 80
KB = KPP - KA
BSTG = 40
EPAD = NS * KPP * CHUNK

NPAD = 10240
WPT = NPAD // NS
NBUF = 2

BN = 512


def _mesh():
    return plsc.VectorSubcoreMesh(
        core_axis_name="c", subcore_axis_name="s", num_cores=NC, num_subcores=NS
    )


def _deg_body(col_hbm, out_hbm, col2d, degv):
    cid = lax.axis_index("c")
    sid = lax.axis_index("s")
    wid = cid * NS + sid

    zeros16 = jnp.zeros((16,), jnp.float32)
    ones16 = jnp.ones((16,), jnp.float32)

    def _fz(r, _):
        degv[pl.ds(r * 16, 16)] = zeros16
        return 0

    lax.fori_loop(0, NPAD // 16, _fz, 0)

    pltpu.sync_copy(col_hbm.at[pl.ds(wid * K3, K3)], col2d)

    def _step(j, _):
        for l in range(CHUNK // 16):
            c16 = col2d[j, pl.ds(l * 16, 16)]
            plsc.addupdate_scatter(degv, [c16], ones16)
        return 0

    lax.fori_loop(0, K3, _step, 0)

    pltpu.sync_copy(degv, out_hbm.at[pl.ds(wid * NPAD, NPAD)])


def _deg_call(col):
    k = pl.kernel(
        _deg_body,
        out_type=jax.ShapeDtypeStruct((NW * NPAD,), jnp.float32),
        mesh=_mesh(),
        compiler_params=pltpu.CompilerParams(needs_layout_passes=False),
        scratch_types=[
            pltpu.VMEM((K3, CHUNK), jnp.int32),
            pltpu.VMEM((NPAD,), jnp.float32),
        ],
    )
    return k(col)


def _agg_body(g_hbm, row_hbm, col_hbm, out_hbm, row2d, col2d, rows_v,
              acc_sh, gsem, ssem):
    cid = lax.axis_index("c")
    sid = lax.axis_index("s")

    zeros16 = jnp.zeros((16,), jnp.float32)

    def _fz(r, _):
        for c in range(D // 16):
            rows_v[r, pl.ds(c * 16, 16)] = zeros16
        return 0

    lax.fori_loop(0, CHUNK, _fz, 0)

    for t in range(WPT // CHUNK):
        pltpu.sync_copy(rows_v.at[pl.ds(0, CHUNK)],
                        acc_sh.at[pl.ds(sid * WPT + t * CHUNK, CHUNK)])
    plsc.subcore_barrier()

    def _gather_start(j):
        pltpu.async_copy(g_hbm.at[row2d.at[j]],
                         rows_v.at[pl.ds((j % NBUF) * CHUNK, CHUNK)], gsem)

    def _gather_wait(j):
        pltpu.make_async_copy(
            g_hbm.at[row2d.at[j]],
            rows_v.at[pl.ds((j % NBUF) * CHUNK, CHUNK)], gsem).wait()

    def _scatter_start(j):
        pltpu.async_copy(rows_v.at[pl.ds((j % NBUF) * CHUNK, CHUNK)],
                         acc_sh.at[col2d.at[j]], ssem, add=True)

    def _scatter_wait(j):
        pltpu.make_async_copy(
            rows_v.at[pl.ds((j % NBUF) * CHUNK, CHUNK)],
            acc_sh.at[col2d.at[j]], ssem).wait()

    is_slow = cid == SLOW_CID
    base = sid * KPP + jnp.where(is_slow, 0, KA)
    nblk = jnp.where(is_slow, KA // BSTG, KB // BSTG)

    def _phase(p, _):
        blk = base + p * BSTG
        pltpu.sync_copy(row_hbm.at[pl.ds(blk, BSTG)], row2d)
        pltpu.sync_copy(col_hbm.at[pl.ds(blk, BSTG)], col2d)

        for m in range(NBUF - 1):
            _gather_start(m)

        def _step(j, _):
            _gather_wait(j)

            @pl.when(j >= 1)
            def _():
                _scatter_wait(j - 1)

            @pl.when(j + NBUF - 1 < BSTG)
            def _():
                _gather_start(j + NBUF - 1)

            _scatter_start(j)
            return 0

        lax.fori_loop(0, BSTG, _step, 0)
        _scatter_wait(BSTG - 1)
        return 0

    lax.fori_loop(0, nblk, _phase, 0)

    plsc.subcore_barrier()

    for t in range(WPT // CHUNK):
        r0 = sid * WPT + t * CHUNK
        pltpu.sync_copy(acc_sh.at[pl.ds(r0, CHUNK)],
                        out_hbm.at[pl.ds(cid * NPAD + r0, CHUNK)])


def _agg_call(g, row, col):
    k = pl.kernel(
        _agg_body,
        out_type=jax.ShapeDtypeStruct((NC * NPAD, D), jnp.float32),
        mesh=_mesh(),
        scratch_types=[
            pltpu.VMEM((BSTG, CHUNK), jnp.int32),
            pltpu.VMEM((BSTG, CHUNK), jnp.int32),
            pltpu.VMEM((NBUF * CHUNK, D), jnp.float32),
            pltpu.VMEM_SHARED((NPAD, D), jnp.float32),
            pltpu.SemaphoreType.DMA,
            pltpu.SemaphoreType.DMA,
        ],
    )
    return k(g, row, col)


def _pre_body(x_ref, w_ref, dt_ref, g_ref, dinv_ref):
    deg = jnp.sum(dt_ref[...], axis=1, keepdims=True) + 1.0
    dinv = lax.rsqrt(deg)
    dinv_ref[...] = jnp.broadcast_to(dinv, dinv_ref.shape)
    h = jnp.dot(x_ref[...], w_ref[...], preferred_element_type=jnp.float32)
    g_ref[...] = h * dinv


def _pre_call(x, w, degT):
    return pl.pallas_call(
        _pre_body,
        grid=(NPAD // BN,),
        in_specs=[
            pl.BlockSpec((BN, D), lambda i: (i, 0)),
            pl.BlockSpec((D, D), lambda i: (0, 0)),
            pl.BlockSpec((BN, NW), lambda i: (i, 0)),
        ],
        out_specs=[
            pl.BlockSpec((BN, D), lambda i: (i, 0)),
            pl.BlockSpec((BN, NW), lambda i: (i, 0)),
        ],
        out_shape=[
            jax.ShapeDtypeStruct((NPAD, D), jnp.float32),
            jax.ShapeDtypeStruct((NPAD, NW), jnp.float32),
        ],
    )(x, w, degT)


def _mid_body(p0_ref, p1_ref, g1_ref, dinv_ref, b_ref, w_ref, g2_ref):
    dinv = dinv_ref[...][:, :1]
    x2 = dinv * (p0_ref[...] + p1_ref[...] + g1_ref[...]) + b_ref[...]
    x2 = jnp.maximum(x2, 0.0)
    g2_ref[...] = (
        jnp.dot(x2, w_ref[...], preferred_element_type=jnp.float32) * dinv
    )


def _mid_call(parts, g1, dinv32, b, w):
    return pl.pallas_call(
        _mid_body,
        grid=(NPAD // BN,),
        in_specs=[
            pl.BlockSpec((BN, D), lambda i: (i, 0)),
            pl.BlockSpec((BN, D), lambda i: (i + NPAD // BN, 0)),
            pl.BlockSpec((BN, D), lambda i: (i, 0)),
            pl.BlockSpec((BN, NW), lambda i: (i, 0)),
            pl.BlockSpec((1, D), lambda i: (0, 0)),
            pl.BlockSpec((D, D), lambda i: (0, 0)),
        ],
        out_specs=pl.BlockSpec((BN, D), lambda i: (i, 0)),
        out_shape=jax.ShapeDtypeStruct((NPAD, D), jnp.float32),
    )(parts, parts, g1, dinv32, b, w)


def _post_body(p0_ref, p1_ref, g2_ref, dinv_ref, b_ref, o_ref):
    dinv = dinv_ref[...][:, :1]
    o_ref[...] = dinv * (p0_ref[...] + p1_ref[...] + g2_ref[...]) + b_ref[...]


def _post_call(parts, g2, dinv32, b):
    return pl.pallas_call(
        _post_body,
        grid=(NPAD // BN,),
        in_specs=[
            pl.BlockSpec((BN, D), lambda i: (i, 0)),
            pl.BlockSpec((BN, D), lambda i: (i + NPAD // BN, 0)),
            pl.BlockSpec((BN, D), lambda i: (i, 0)),
            pl.BlockSpec((BN, NW), lambda i: (i, 0)),
            pl.BlockSpec((1, D), lambda i: (0, 0)),
        ],
        out_specs=pl.BlockSpec((BN, D), lambda i: (i, 0)),
        out_shape=jax.ShapeDtypeStruct((NPAD, D), jnp.float32),
    )(parts, parts, g2, dinv32, b)


def kernel(node_features, edge_index, W1, b1, W2, b2):
    ei = edge_index.astype(jnp.int32)
    padv = N + (jnp.arange(EPAD - E, dtype=jnp.int32) % (NPAD - N))
    row = jnp.concatenate([ei[0], padv]).reshape(NW * K3, CHUNK)
    col = jnp.concatenate([ei[1], padv]).reshape(NW * K3, CHUNK)
    x_pad = jnp.pad(node_features, ((0, NPAD - N), (0, 0)))

    deg = _deg_call(col)
    degT = deg.reshape(NW, NPAD).T

    g1, dinv32 = _pre_call(x_pad, W1, degT)
    agg1 = _agg_call(g1, row, col)
    g2 = _mid_call(agg1, g1, dinv32, b1.reshape(1, D), W2)
    agg2 = _agg_call(g2, row, col)
    out = _post_call(agg2, g2, dinv32, b2.reshape(1, D))
    return out[:N]
